# baseline (speedup 1.0000x reference)
import jax
from jax.experimental import pallas as pl
from jax.experimental.pallas import tpu as pltpu


def _copy_body(f_in, ei_in, ea_in, f_out, ei_out, ea_out, sem):
    copies = []
    for i, (src, dst) in enumerate(((f_in, f_out), (ei_in, ei_out), (ea_in, ea_out))):
        rows, minor = src.shape
        grp = 2500
        flat_src = src.reshape(rows // grp, grp, minor)
        flat_dst = dst.reshape(rows // grp, grp, minor)
        copies.append(pltpu.make_async_copy(flat_src, flat_dst, sem.at[i]))
    for c in copies:
        c.start()
    for c in copies:
        c.wait()


def kernel(feats, edge_index, edge_attr):
    return pl.pallas_call(
        _copy_body,
        in_specs=[pl.BlockSpec(memory_space=pl.ANY)] * 3,
        out_specs=[pl.BlockSpec(memory_space=pl.ANY)] * 3,
        out_shape=[
            jax.ShapeDtypeStruct(feats.shape, feats.dtype),
            jax.ShapeDtypeStruct(edge_index.shape, edge_index.dtype),
            jax.ShapeDtypeStruct(edge_attr.shape, edge_attr.dtype),
        ],
        scratch_shapes=[pltpu.SemaphoreType.DMA((3,))],
    )(feats, edge_index, edge_attr)


# speedup vs baseline: 18.0372x; 18.0372x over previous
import functools

import jax
from jax import lax
from jax.experimental import pallas as pl
from jax.experimental.pallas import tpu as pltpu
from jax.experimental.pallas import tpu_sc as plsc

_CHUNK = 200


def _feats_body(f_in, f_out):
    f_out[...] = f_in[...]


def _copy_feats(feats):
    n, d = feats.shape
    grid = 5
    return pl.pallas_call(
        _feats_body,
        grid=(grid,),
        in_specs=[pl.BlockSpec((n // grid, d), lambda i: (i, 0))],
        out_specs=pl.BlockSpec((n // grid, d), lambda i: (i, 0)),
        out_shape=jax.ShapeDtypeStruct(feats.shape, feats.dtype),
        compiler_params=pltpu.CompilerParams(
            dimension_semantics=("arbitrary",),
        ),
    )(feats)


def _make_sc_copy(e, ik, ak, ei_dtype, ea_dtype):
    info = plsc.get_sparse_core_info()
    nc, ns = info.num_cores, info.num_subcores
    nw = nc * ns
    rows_per_w = e // nw
    nchunks = rows_per_w // _CHUNK
    mesh = plsc.VectorSubcoreMesh(core_axis_name="c", subcore_axis_name="s")

    @functools.partial(
        pl.kernel,
        mesh=mesh,
        out_type=[
            jax.ShapeDtypeStruct((e, ik), ei_dtype),
            jax.ShapeDtypeStruct((e, ak), ea_dtype),
        ],
        scratch_types=[
            pltpu.VMEM((_CHUNK, ik), ei_dtype),
            pltpu.VMEM((_CHUNK, ik), ei_dtype),
            pltpu.VMEM((_CHUNK, ak), ea_dtype),
            pltpu.VMEM((_CHUNK, ak), ea_dtype),
            pltpu.SemaphoreType.DMA((2, 2)),
            pltpu.SemaphoreType.DMA((2, 2)),
        ],
    )
    def sc_copy(ei_hbm, ea_hbm, ei_out, ea_out, ei_v0, ei_v1, ea_v0, ea_v1, in_sem, out_sem):
        ei_v = (ei_v0, ei_v1)
        ea_v = (ea_v0, ea_v1)
        wid = lax.axis_index("s") * nc + lax.axis_index("c")
        base = wid * rows_per_w

        def start_in(j, buf):
            o = base + j * _CHUNK
            pltpu.async_copy(ei_hbm.at[pl.ds(o, _CHUNK)], ei_v[buf], in_sem.at[buf, 0])
            pltpu.async_copy(ea_hbm.at[pl.ds(o, _CHUNK)], ea_v[buf], in_sem.at[buf, 1])

        def wait_in(buf):
            pltpu.make_async_copy(ei_hbm.at[pl.ds(base, _CHUNK)], ei_v[buf], in_sem.at[buf, 0]).wait()
            pltpu.make_async_copy(ea_hbm.at[pl.ds(base, _CHUNK)], ea_v[buf], in_sem.at[buf, 1]).wait()

        def start_out(j, buf):
            o = base + j * _CHUNK
            pltpu.async_copy(ei_v[buf], ei_out.at[pl.ds(o, _CHUNK)], out_sem.at[buf, 0])
            pltpu.async_copy(ea_v[buf], ea_out.at[pl.ds(o, _CHUNK)], out_sem.at[buf, 1])

        def wait_out(buf):
            pltpu.make_async_copy(ei_v[buf], ei_out.at[pl.ds(base, _CHUNK)], out_sem.at[buf, 0]).wait()
            pltpu.make_async_copy(ea_v[buf], ea_out.at[pl.ds(base, _CHUNK)], out_sem.at[buf, 1]).wait()

        start_in(0, 0)
        start_in(1, 1)

        @pl.loop(0, nchunks - 2, step=2)
        def _body(g):
            wait_in(0)
            start_out(g, 0)
            wait_in(1)
            start_out(g + 1, 1)
            wait_out(0)
            start_in(g + 2, 0)
            wait_out(1)
            start_in(g + 3, 1)

        wait_in(0)
        start_out(nchunks - 2, 0)
        wait_in(1)
        start_out(nchunks - 1, 1)
        wait_out(0)
        wait_out(1)

    return sc_copy


def kernel(feats, edge_index, edge_attr):
    e, ik = edge_index.shape
    _, ak = edge_attr.shape
    sc_copy = _make_sc_copy(e, ik, ak, edge_index.dtype, edge_attr.dtype)
    ei_o, ea_o = sc_copy(edge_index, edge_attr)
    f_o = _copy_feats(feats)
    return (f_o, ei_o, ea_o)


# speedup vs baseline: 18.3101x; 1.0151x over previous
import functools

import jax
from jax import lax
from jax.experimental import pallas as pl
from jax.experimental.pallas import tpu as pltpu
from jax.experimental.pallas import tpu_sc as plsc


def _feats_body(f_in, f_out):
    f_out[...] = f_in[...]


def _copy_feats(feats):
    n, d = feats.shape
    grid = 5
    return pl.pallas_call(
        _feats_body,
        grid=(grid,),
        in_specs=[pl.BlockSpec((n // grid, d), lambda i: (i, 0))],
        out_specs=pl.BlockSpec((n // grid, d), lambda i: (i, 0)),
        out_shape=jax.ShapeDtypeStruct(feats.shape, feats.dtype),
        compiler_params=pltpu.CompilerParams(
            dimension_semantics=("arbitrary",),
        ),
    )(feats)


def _make_sc_copy(ei_shape, ea_shape, ei_dtype, ea_dtype, nc, ns):
    nw = nc * ns
    ei_rows_w = ei_shape[0] // nw
    ea_rows_w = ea_shape[0] // nw
    mesh = plsc.VectorSubcoreMesh(core_axis_name="c", subcore_axis_name="s")

    @functools.partial(
        pl.kernel,
        mesh=mesh,
        out_type=[
            jax.ShapeDtypeStruct(ei_shape, ei_dtype),
            jax.ShapeDtypeStruct(ea_shape, ea_dtype),
        ],
        scratch_types=[
            pltpu.VMEM((1, ei_shape[1]), ei_dtype),
            pltpu.VMEM((1, ei_shape[1]), ei_dtype),
            pltpu.VMEM((1, ea_shape[1]), ea_dtype),
            pltpu.VMEM((1, ea_shape[1]), ea_dtype),
            pltpu.SemaphoreType.DMA((2, 2)),
            pltpu.SemaphoreType.DMA((2, 2)),
        ],
    )
    def sc_copy(ei_hbm, ea_hbm, ei_out, ea_out, ei_v0, ei_v1, ea_v0, ea_v1, in_sem, out_sem):
        wid = lax.axis_index("s") * nc + lax.axis_index("c")

        def copy_array(src, dst, bufs, rows_w, arr):
            base = wid * rows_w

            def start_in(j, b):
                pltpu.async_copy(src.at[pl.ds(base + j, 1)], bufs[b], in_sem.at[b, arr])

            def wait_in(b):
                pltpu.make_async_copy(src.at[pl.ds(base, 1)], bufs[b], in_sem.at[b, arr]).wait()

            def start_out(j, b):
                pltpu.async_copy(bufs[b], dst.at[pl.ds(base + j, 1)], out_sem.at[b, arr])

            def wait_out(b):
                pltpu.make_async_copy(bufs[b], dst.at[pl.ds(base, 1)], out_sem.at[b, arr]).wait()

            start_in(0, 0)
            if rows_w > 1:
                start_in(1, 1)
            for j in range(rows_w):
                b = j % 2
                wait_in(b)
                start_out(j, b)
                if j + 2 < rows_w:
                    wait_out(b)
                    start_in(j + 2, b)
            wait_out((rows_w - 1) % 2)
            if rows_w > 1:
                wait_out(rows_w % 2)

        copy_array(ei_hbm, ei_out, (ei_v0, ei_v1), ei_rows_w, 0)
        copy_array(ea_hbm, ea_out, (ea_v0, ea_v1), ea_rows_w, 1)

    return sc_copy


def kernel(feats, edge_index, edge_attr):
    e, ik = edge_index.shape
    _, ak = edge_attr.shape

    ei2 = edge_index.reshape(64, (e * ik) // 64)
    ea2 = edge_attr.reshape(320, (e * ak) // 320)

    info = plsc.get_sparse_core_info()
    sc_copy = _make_sc_copy(ei2.shape, ea2.shape, ei2.dtype, ea2.dtype,
                            info.num_cores, info.num_subcores)
    ei_o, ea_o = sc_copy(ei2, ea2)
    f_o = _copy_feats(feats)
    return (f_o, ei_o.reshape(e, ik), ea_o.reshape(e, ak))
